# baseline (speedup 1.0000x reference)
import functools

import jax
import jax.numpy as jnp
from jax import lax
from jax.experimental import pallas as pl
from jax.experimental.pallas import tpu as pltpu
from jax.experimental.pallas import tpu_sc as plsc

_B = 16384
_V = 1000000
_SED = 32
_PED = 64
_NMODE_PAD = 16
_PACK = 8
_WORDS = 16
_LANES = _PACK * _WORDS

_TBLK = 131072
_TSUB = _TBLK // _PACK
_TSTEPS = -(-_V // _TBLK)

_NW = 32
_ROWS_W = _B // _NW
_CHUNK = 128
_NCHUNK = _ROWS_W // _CHUNK

_BLK = 8192


def _rne16(u):
    return (u + 0x7FFF + ((u >> 16) & 1)) >> 16


def _repack_body(in_ref, out_ref):
    x = in_ref[...]
    xu = lax.bitcast_convert_type(x, jnp.uint32)
    lo = _rne16(xu[:_WORDS, :])
    hi = _rne16(xu[_WORDS:, :])
    pw = lax.bitcast_convert_type(lo | (hi << 16), jnp.float32)
    stacked = jnp.concatenate(
        [pw[:, a * _TSUB:(a + 1) * _TSUB] for a in range(_PACK)], axis=0)
    out_ref[...] = jnp.transpose(stacked)


def _tc_repack(ut_t):
    return pl.pallas_call(
        _repack_body,
        grid=(_TSTEPS,),
        in_specs=[pl.BlockSpec((_SED, _TBLK), lambda i: (0, i))],
        out_specs=pl.BlockSpec((_TSUB, _LANES), lambda i: (i, 0)),
        out_shape=jax.ShapeDtypeStruct((_TSTEPS * _TSUB, _LANES), jnp.float32),
    )(ut_t)


def _sc_gather(table128, idx2d):
    mesh = plsc.VectorSubcoreMesh(core_axis_name="c", subcore_axis_name="s")

    @functools.partial(
        pl.kernel,
        mesh=mesh,
        out_type=jax.ShapeDtypeStruct((_B, _LANES), jnp.float32),
        scratch_types=[
            pltpu.VMEM((_NCHUNK, _CHUNK), jnp.int32),
            pltpu.VMEM((_NCHUNK, _CHUNK, _LANES), jnp.float32),
            pltpu.SemaphoreType.DMA,
        ],
    )
    def gather_kernel(table_hbm, idx_hbm, out_hbm, idx_v, rows_v, sem):
        wid = lax.axis_index("s") * 2 + lax.axis_index("c")
        r0 = wid * _NCHUNK
        pltpu.sync_copy(idx_hbm.at[pl.ds(r0, _NCHUNK)], idx_v)
        copies = [
            pltpu.async_copy(table_hbm.at[idx_v.at[j]], rows_v.at[j], sem)
            for j in range(_NCHUNK)
        ]
        for c in copies:
            c.wait()
        for j in range(_NCHUNK):
            pltpu.sync_copy(
                rows_v.at[j], out_hbm.at[pl.ds((r0 + j) * _CHUNK, _CHUNK)]
            )

    return gather_kernel(table128, idx2d)


def _tc_body(ug_ref, grp_ref, mode_ref, ts_ref, mt_ref, wlo_ref, whi_ref,
             wm_ref, wti_ref, wt_ref, bt_ref, bp_ref, out_ref):
    mo = jnp.dot(mt_ref[...], wm_ref[...], preferred_element_type=jnp.float32)
    wc = jnp.dot(wti_ref[...], wt_ref[...], preferred_element_type=jnp.float32)
    bias = (
        jnp.dot(bt_ref[...], wt_ref[...], preferred_element_type=jnp.float32)
        + bp_ref[...]
    )
    grp_col = jnp.transpose(grp_ref[...])
    mode_col = jnp.transpose(mode_ref[...])
    lane_grp = lax.broadcasted_iota(jnp.int32, (_BLK, _LANES), 1) // _WORDS
    xu = jnp.where(lane_grp == grp_col,
                   lax.bitcast_convert_type(ug_ref[...], jnp.uint32),
                   jnp.uint32(0))
    mlo = lax.bitcast_convert_type(xu << 16, jnp.float32).astype(jnp.bfloat16)
    mhi = lax.bitcast_convert_type(
        xu & jnp.uint32(0xFFFF0000), jnp.float32).astype(jnp.bfloat16)
    acc = lax.dot_general(wlo_ref[...], mlo, (((0,), (1,)), ((), ())),
                          preferred_element_type=jnp.float32)
    acc = acc + lax.dot_general(whi_ref[...], mhi, (((0,), (1,)), ((), ())),
                                preferred_element_type=jnp.float32)
    iota = lax.broadcasted_iota(jnp.int32, (_BLK, _NMODE_PAD), 1)
    oh = (mode_col == iota).astype(jnp.float32)
    acc = acc + lax.dot_general(mo, oh, (((0,), (1,)), ((), ())),
                                preferred_element_type=jnp.float32)
    acc = acc + lax.dot_general(wc, ts_ref[...], (((0,), (0,)), ((), ())),
                                preferred_element_type=jnp.float32)
    out_ref[...] = acc + jnp.transpose(bias)


def _tc_dense(ug128, grp_t, mode_t, ts_t, mt16, Wlo8, Whi8, Wm, W_time, Wt,
              bt2d, bp2d):
    return pl.pallas_call(
        _tc_body,
        grid=(_B // _BLK,),
        in_specs=[
            pl.BlockSpec((_BLK, _LANES), lambda i: (i, 0)),
            pl.BlockSpec((1, _BLK), lambda i: (0, i)),
            pl.BlockSpec((1, _BLK), lambda i: (0, i)),
            pl.BlockSpec((6, _BLK), lambda i: (0, i)),
            pl.BlockSpec((_NMODE_PAD, _SED), lambda i: (0, 0)),
            pl.BlockSpec((_LANES, _PED), lambda i: (0, 0)),
            pl.BlockSpec((_LANES, _PED), lambda i: (0, 0)),
            pl.BlockSpec((_SED, _PED), lambda i: (0, 0)),
            pl.BlockSpec((6, _SED), lambda i: (0, 0)),
            pl.BlockSpec((_SED, _PED), lambda i: (0, 0)),
            pl.BlockSpec((1, _SED), lambda i: (0, 0)),
            pl.BlockSpec((1, _PED), lambda i: (0, 0)),
        ],
        out_specs=pl.BlockSpec((_PED, _BLK), lambda i: (0, i)),
        out_shape=jax.ShapeDtypeStruct((_PED, _B), jnp.float32),
    )(ug128, grp_t, mode_t, ts_t, mt16, Wlo8, Whi8, Wm, W_time, Wt, bt2d,
      bp2d)


def kernel(user_id, transport_mode, timestamp, user_table, mode_table,
           W_time, b_time, W_pref, b_pref):
    uid = user_id.astype(jnp.int32)
    table128 = _tc_repack(user_table.T)
    rows = (uid // _TBLK) * _TSUB + (uid % _TSUB)
    grp = (uid % _TBLK) // _TSUB
    ug128 = _sc_gather(table128, rows.reshape(_B // _CHUNK, _CHUNK))
    mt16 = jnp.zeros((_NMODE_PAD, _SED), jnp.float32).at[:12].set(mode_table)
    Wu = W_pref[:_SED]
    Wm = W_pref[_SED:2 * _SED]
    Wt = W_pref[2 * _SED:]
    Wlo8 = jnp.tile(Wu[:_WORDS], (_PACK, 1)).astype(jnp.bfloat16)
    Whi8 = jnp.tile(Wu[_WORDS:], (_PACK, 1)).astype(jnp.bfloat16)
    mode_t = transport_mode.astype(jnp.int32).reshape(1, _B)
    out_t = _tc_dense(ug128, grp.reshape(1, _B), mode_t, timestamp.T, mt16,
                      Wlo8, Whi8, Wm, W_time, Wt, b_time.reshape(1, _SED),
                      b_pref.reshape(1, _PED))
    return out_t.T


# speedup vs baseline: 1.0027x; 1.0027x over previous
import functools

import jax
import jax.numpy as jnp
from jax import lax
from jax.experimental import pallas as pl
from jax.experimental.pallas import tpu as pltpu
from jax.experimental.pallas import tpu_sc as plsc

_B = 16384
_V = 1000000
_SED = 32
_PED = 64
_NMODE_PAD = 16
_PACK = 8
_WORDS = 16
_LANES = _PACK * _WORDS

_TBLK = 131072
_TSUB = _TBLK // _PACK
_TSTEPS = -(-_V // _TBLK)

_NW = 32
_ROWS_W = _B // _NW
_CHUNK = 128
_NCHUNK = _ROWS_W // _CHUNK

_BLK = 4096


def _rne16(u):
    return (u + 0x7FFF + ((u >> 16) & 1)) >> 16


def _repack_body(in_ref, out_ref):
    x = in_ref[...]
    xu = lax.bitcast_convert_type(x, jnp.uint32)
    lo = _rne16(xu[:_WORDS, :])
    hi = _rne16(xu[_WORDS:, :])
    pw = lax.bitcast_convert_type(lo | (hi << 16), jnp.float32)
    stacked = jnp.concatenate(
        [pw[:, a * _TSUB:(a + 1) * _TSUB] for a in range(_PACK)], axis=0)
    out_ref[...] = jnp.transpose(stacked)


def _tc_repack(ut_t):
    return pl.pallas_call(
        _repack_body,
        grid=(_TSTEPS,),
        in_specs=[pl.BlockSpec((_SED, _TBLK), lambda i: (0, i))],
        out_specs=pl.BlockSpec((_TSUB, _LANES), lambda i: (i, 0)),
        out_shape=jax.ShapeDtypeStruct((_TSTEPS * _TSUB, _LANES), jnp.float32),
    )(ut_t)


def _sc_gather(table128, idx2d):
    mesh = plsc.VectorSubcoreMesh(core_axis_name="c", subcore_axis_name="s")

    @functools.partial(
        pl.kernel,
        mesh=mesh,
        out_type=jax.ShapeDtypeStruct((_B, _LANES), jnp.float32),
        scratch_types=[
            pltpu.VMEM((_NCHUNK, _CHUNK), jnp.int32),
            pltpu.VMEM((_NCHUNK, _CHUNK, _LANES), jnp.float32),
            pltpu.SemaphoreType.DMA,
        ],
    )
    def gather_kernel(table_hbm, idx_hbm, out_hbm, idx_v, rows_v, sem):
        wid = lax.axis_index("s") * 2 + lax.axis_index("c")
        r0 = wid * _NCHUNK
        pltpu.sync_copy(idx_hbm.at[pl.ds(r0, _NCHUNK)], idx_v)
        copies = [
            pltpu.async_copy(table_hbm.at[idx_v.at[j]], rows_v.at[j], sem)
            for j in range(_NCHUNK)
        ]
        for c in copies:
            c.wait()
        for j in range(_NCHUNK):
            pltpu.sync_copy(
                rows_v.at[j], out_hbm.at[pl.ds((r0 + j) * _CHUNK, _CHUNK)]
            )

    return gather_kernel(table128, idx2d)


def _tc_body(ug_ref, grp_ref, mode_ref, ts_ref, mt_ref, wlo_ref, whi_ref,
             wm_ref, wti_ref, wt_ref, bt_ref, bp_ref, out_ref):
    mo = jnp.dot(mt_ref[...], wm_ref[...], preferred_element_type=jnp.float32)
    wc = jnp.dot(wti_ref[...], wt_ref[...], preferred_element_type=jnp.float32)
    bias = (
        jnp.dot(bt_ref[...], wt_ref[...], preferred_element_type=jnp.float32)
        + bp_ref[...]
    )
    grp_col = jnp.transpose(grp_ref[...])
    mode_col = jnp.transpose(mode_ref[...])
    lane_grp = lax.broadcasted_iota(jnp.int32, (_BLK, _LANES), 1) // _WORDS
    xu = jnp.where(lane_grp == grp_col,
                   lax.bitcast_convert_type(ug_ref[...], jnp.uint32),
                   jnp.uint32(0))
    mlo = lax.bitcast_convert_type(xu << 16, jnp.float32).astype(jnp.bfloat16)
    mhi = lax.bitcast_convert_type(
        xu & jnp.uint32(0xFFFF0000), jnp.float32).astype(jnp.bfloat16)
    acc = lax.dot_general(wlo_ref[...], mlo, (((0,), (1,)), ((), ())),
                          preferred_element_type=jnp.float32)
    acc = acc + lax.dot_general(whi_ref[...], mhi, (((0,), (1,)), ((), ())),
                                preferred_element_type=jnp.float32)
    iota = lax.broadcasted_iota(jnp.int32, (_BLK, _NMODE_PAD), 1)
    oh = (mode_col == iota).astype(jnp.float32)
    acc = acc + lax.dot_general(mo, oh, (((0,), (1,)), ((), ())),
                                preferred_element_type=jnp.float32)
    acc = acc + lax.dot_general(wc, ts_ref[...], (((0,), (0,)), ((), ())),
                                preferred_element_type=jnp.float32)
    out_ref[...] = acc + jnp.transpose(bias)


def _tc_dense(ug128, grp_t, mode_t, ts_t, mt16, Wlo8, Whi8, Wm, W_time, Wt,
              bt2d, bp2d):
    return pl.pallas_call(
        _tc_body,
        grid=(_B // _BLK,),
        in_specs=[
            pl.BlockSpec((_BLK, _LANES), lambda i: (i, 0)),
            pl.BlockSpec((1, _BLK), lambda i: (0, i)),
            pl.BlockSpec((1, _BLK), lambda i: (0, i)),
            pl.BlockSpec((6, _BLK), lambda i: (0, i)),
            pl.BlockSpec((_NMODE_PAD, _SED), lambda i: (0, 0)),
            pl.BlockSpec((_LANES, _PED), lambda i: (0, 0)),
            pl.BlockSpec((_LANES, _PED), lambda i: (0, 0)),
            pl.BlockSpec((_SED, _PED), lambda i: (0, 0)),
            pl.BlockSpec((6, _SED), lambda i: (0, 0)),
            pl.BlockSpec((_SED, _PED), lambda i: (0, 0)),
            pl.BlockSpec((1, _SED), lambda i: (0, 0)),
            pl.BlockSpec((1, _PED), lambda i: (0, 0)),
        ],
        out_specs=pl.BlockSpec((_PED, _BLK), lambda i: (0, i)),
        out_shape=jax.ShapeDtypeStruct((_PED, _B), jnp.float32),
    )(ug128, grp_t, mode_t, ts_t, mt16, Wlo8, Whi8, Wm, W_time, Wt, bt2d,
      bp2d)


def kernel(user_id, transport_mode, timestamp, user_table, mode_table,
           W_time, b_time, W_pref, b_pref):
    uid = user_id.astype(jnp.int32)
    table128 = _tc_repack(user_table.T)
    rows = (uid // _TBLK) * _TSUB + (uid % _TSUB)
    grp = (uid % _TBLK) // _TSUB
    ug128 = _sc_gather(table128, rows.reshape(_B // _CHUNK, _CHUNK))
    mt16 = jnp.zeros((_NMODE_PAD, _SED), jnp.float32).at[:12].set(mode_table)
    Wu = W_pref[:_SED]
    Wm = W_pref[_SED:2 * _SED]
    Wt = W_pref[2 * _SED:]
    Wlo8 = jnp.tile(Wu[:_WORDS], (_PACK, 1)).astype(jnp.bfloat16)
    Whi8 = jnp.tile(Wu[_WORDS:], (_PACK, 1)).astype(jnp.bfloat16)
    mode_t = transport_mode.astype(jnp.int32).reshape(1, _B)
    out_t = _tc_dense(ug128, grp.reshape(1, _B), mode_t, timestamp.T, mt16,
                      Wlo8, Whi8, Wm, W_time, Wt, b_time.reshape(1, _SED),
                      b_pref.reshape(1, _PED))
    return out_t.T
